# baseline (speedup 1.0000x reference)
import functools

import jax
import jax.numpy as jnp
import numpy as np
from jax import lax
from jax.experimental import pallas as pl
from jax.experimental.pallas import tpu as pltpu
from jax.experimental.pallas import tpu_sc as plsc

VOCAB = 100000
HIDDEN = 768
SEQ = 2048
BATCH = 4
SCALE = float(np.sqrt(HIDDEN))

NW = 32
S_W = SEQ // NW
S_C = 8
R_C = BATCH * S_C
NCHUNK = S_W // S_C
NV = HIDDEN // 16
NBUF = 2


def _sc_embed(ids, table, pos_emb):
    mesh = plsc.VectorSubcoreMesh(core_axis_name="c", subcore_axis_name="s")

    @functools.partial(
        pl.kernel,
        out_type=jax.ShapeDtypeStruct((BATCH, SEQ, HIDDEN), jnp.float32),
        mesh=mesh,
        scratch_types=[
            pltpu.VMEM((BATCH, S_W), jnp.int32),
            pltpu.VMEM((S_W * BATCH,), jnp.int32),
            pltpu.VMEM((NBUF, R_C, HIDDEN), jnp.float32),
            pltpu.VMEM((NBUF, S_C, HIDDEN), jnp.float32),
            pltpu.VMEM((NBUF, BATCH, S_C, HIDDEN), jnp.float32),
            pltpu.SemaphoreType.DMA,
            pltpu.SemaphoreType.DMA,
            pltpu.SemaphoreType.DMA,
        ],
    )
    def k(ids_hbm, table_hbm, pos_hbm, out_hbm, idx_v, idx_r, rows_v, pos_v,
          res_v, sem_g, sem_p, sem_o):
        wid = lax.axis_index("s") * 2 + lax.axis_index("c")
        s_base = wid * S_W

        for b in range(BATCH):
            pltpu.sync_copy(
                ids_hbm.at[pl.ds(b * SEQ + s_base, S_W)], idx_v.at[b]
            )

        def issue_in(t, bf):
            for b in range(BATCH):
                pltpu.async_copy(
                    table_hbm.at[idx_v.at[b, pl.ds(t * S_C, S_C)]],
                    rows_v.at[bf, pl.ds(b * S_C, S_C)], sem_g,
                )
            pltpu.async_copy(
                pos_hbm.at[pl.ds(s_base + t * S_C, S_C)], pos_v.at[bf], sem_p
            )

        for bf in range(NBUF):
            issue_in(bf, bf)

        def outer(g, _):
            for bf in range(NBUF):
                t = NBUF * g + bf
                pltpu.make_async_copy(
                    table_hbm.at[pl.ds(0, R_C)], rows_v.at[bf], sem_g
                ).wait()
                pltpu.make_async_copy(
                    pos_hbm.at[pl.ds(0, S_C)], pos_v.at[bf], sem_p
                ).wait()
                @pl.when(t >= NBUF)
                def _():
                    pltpu.make_async_copy(
                        res_v.at[bf], out_hbm.at[:, pl.ds(0, S_C)], sem_o
                    ).wait()

                def row_body(i, _):
                    @plsc.parallel_loop(0, HIDDEN, 16, unroll=2)
                    def _(o):
                        sl = pl.ds(pl.multiple_of(o, 16), 16)
                        p = pos_v[bf, i, sl]
                        for b in range(BATCH):
                            res_v[bf, b, i, sl] = (
                                rows_v[bf, b * S_C + i, sl] * SCALE + p
                            )

                    return 0

                lax.fori_loop(0, S_C, row_body, 0)

                pltpu.async_copy(
                    res_v.at[bf],
                    out_hbm.at[:, pl.ds(s_base + t * S_C, S_C)], sem_o,
                )

                @pl.when(t + NBUF < NCHUNK)
                def _():
                    issue_in(t + NBUF, bf)

            return 0

        lax.fori_loop(0, NCHUNK // NBUF, outer, 0)

        for bf in range(NBUF):
            pltpu.make_async_copy(
                res_v.at[bf], out_hbm.at[:, pl.ds(0, S_C)], sem_o
            ).wait()

    return k(ids, table, pos_emb)


def kernel(input_ids, table, pos_emb):
    ids = input_ids.reshape(-1).astype(jnp.int32)
    return _sc_embed(ids, table, pos_emb)


# speedup vs baseline: 1.0286x; 1.0286x over previous
import functools

import jax
import jax.numpy as jnp
import numpy as np
from jax import lax
from jax.experimental import pallas as pl
from jax.experimental.pallas import tpu as pltpu
from jax.experimental.pallas import tpu_sc as plsc

VOCAB = 100000
HIDDEN = 768
SEQ = 2048
BATCH = 4
SCALE = float(np.sqrt(HIDDEN))

NW = 32
S_W = SEQ // NW
S_C = 8
R_C = BATCH * S_C
NCHUNK = S_W // S_C
NV = HIDDEN // 16
NBUF = 2


def _sc_embed(ids, table, pos_emb):
    mesh = plsc.VectorSubcoreMesh(core_axis_name="c", subcore_axis_name="s")

    @functools.partial(
        pl.kernel,
        out_type=jax.ShapeDtypeStruct((BATCH, SEQ, HIDDEN), jnp.float32),
        mesh=mesh,
        scratch_types=[
            pltpu.VMEM((BATCH, S_W), jnp.int32),
            pltpu.VMEM((S_W * BATCH,), jnp.int32),
            pltpu.VMEM((NBUF, R_C, HIDDEN), jnp.float32),
            pltpu.VMEM((NBUF, S_C, HIDDEN), jnp.float32),
            pltpu.VMEM((NBUF, BATCH, S_C, HIDDEN), jnp.float32),
            pltpu.SemaphoreType.DMA,
            pltpu.SemaphoreType.DMA,
            pltpu.SemaphoreType.DMA,
        ],
    )
    def k(ids_hbm, table_hbm, pos_hbm, out_hbm, idx_v, idx_r, rows_v, pos_v,
          res_v, sem_g, sem_p, sem_o):
        wid = lax.axis_index("s") * 2 + lax.axis_index("c")
        s_base = wid * S_W

        for b in range(BATCH):
            pltpu.async_copy(
                ids_hbm.at[pl.ds(b * SEQ + s_base, S_W)], idx_v.at[b], sem_g
            )
        for b in range(BATCH):
            pltpu.make_async_copy(
                ids_hbm.at[pl.ds(0, S_W)], idx_v.at[b], sem_g
            ).wait()

        def issue_in(t, bf):
            for b in range(BATCH):
                pltpu.async_copy(
                    table_hbm.at[idx_v.at[b, pl.ds(t * S_C, S_C)]],
                    rows_v.at[bf, pl.ds(b * S_C, S_C)], sem_g,
                )
            pltpu.async_copy(
                pos_hbm.at[pl.ds(s_base + t * S_C, S_C)], pos_v.at[bf], sem_p
            )

        for bf in range(NBUF):
            issue_in(bf, bf)

        def outer(g, _):
            for bf in range(NBUF):
                t = NBUF * g + bf
                pltpu.make_async_copy(
                    table_hbm.at[pl.ds(0, R_C)], rows_v.at[bf], sem_g
                ).wait()
                pltpu.make_async_copy(
                    pos_hbm.at[pl.ds(0, S_C)], pos_v.at[bf], sem_p
                ).wait()
                @pl.when(t >= NBUF)
                def _():
                    pltpu.make_async_copy(
                        res_v.at[bf], out_hbm.at[:, pl.ds(0, S_C)], sem_o
                    ).wait()

                def row_body(i, _):
                    @plsc.parallel_loop(0, HIDDEN, 16, unroll=2)
                    def _(o):
                        sl = pl.ds(pl.multiple_of(o, 16), 16)
                        p = pos_v[bf, i, sl]
                        for b in range(BATCH):
                            res_v[bf, b, i, sl] = (
                                rows_v[bf, b * S_C + i, sl] * SCALE + p
                            )

                    return 0

                lax.fori_loop(0, S_C, row_body, 0)

                pltpu.async_copy(
                    res_v.at[bf],
                    out_hbm.at[:, pl.ds(s_base + t * S_C, S_C)], sem_o,
                )

                @pl.when(t + NBUF < NCHUNK)
                def _():
                    issue_in(t + NBUF, bf)

            return 0

        lax.fori_loop(0, NCHUNK // NBUF, outer, 0)

        for bf in range(NBUF):
            pltpu.make_async_copy(
                res_v.at[bf], out_hbm.at[:, pl.ds(0, S_C)], sem_o
            ).wait()

    return k(ids, table, pos_emb)


def kernel(input_ids, table, pos_emb):
    ids = input_ids.reshape(-1).astype(jnp.int32)
    return _sc_embed(ids, table, pos_emb)


# speedup vs baseline: 1.0313x; 1.0026x over previous
import functools

import jax
import jax.numpy as jnp
import numpy as np
from jax import lax
from jax.experimental import pallas as pl
from jax.experimental.pallas import tpu as pltpu
from jax.experimental.pallas import tpu_sc as plsc

VOCAB = 100000
HIDDEN = 768
SEQ = 2048
BATCH = 4
SCALE = float(np.sqrt(HIDDEN))

NW = 32
S_W = SEQ // NW
S_C = 8
R_C = BATCH * S_C
NCHUNK = S_W // S_C
NV = HIDDEN // 16
NBUF = 2


def _sc_embed(ids, table, pos_emb):
    mesh = plsc.VectorSubcoreMesh(core_axis_name="c", subcore_axis_name="s")

    @functools.partial(
        pl.kernel,
        out_type=jax.ShapeDtypeStruct((BATCH, SEQ, HIDDEN), jnp.float32),
        mesh=mesh,
        scratch_types=[
            pltpu.VMEM((S_W * BATCH,), jnp.int32),
            pltpu.VMEM((NBUF, R_C, HIDDEN), jnp.float32),
            pltpu.VMEM((NBUF, S_C, HIDDEN), jnp.float32),
            pltpu.VMEM((NBUF, BATCH, S_C, HIDDEN), jnp.float32),
            pltpu.SemaphoreType.DMA,
            pltpu.SemaphoreType.DMA,
            pltpu.SemaphoreType.DMA,
        ],
    )
    def k(ids_hbm, table_hbm, pos_hbm, out_hbm, idx_r, rows_v, pos_v,
          res_v, sem_g, sem_p, sem_o):
        wid = lax.axis_index("s") * 2 + lax.axis_index("c")
        s_base = wid * S_W

        pltpu.sync_copy(
            ids_hbm.at[pl.ds(wid * (S_W * BATCH), S_W * BATCH)], idx_r
        )

        def issue_in(t, bf):
            pltpu.async_copy(
                table_hbm.at[idx_r.at[pl.ds(t * R_C, R_C)]], rows_v.at[bf],
                sem_g,
            )
            pltpu.async_copy(
                pos_hbm.at[pl.ds(s_base + t * S_C, S_C)], pos_v.at[bf], sem_p
            )

        for bf in range(NBUF):
            issue_in(bf, bf)

        def outer(g, _):
            for bf in range(NBUF):
                t = NBUF * g + bf
                pltpu.make_async_copy(
                    table_hbm.at[pl.ds(0, R_C)], rows_v.at[bf], sem_g
                ).wait()
                pltpu.make_async_copy(
                    pos_hbm.at[pl.ds(0, S_C)], pos_v.at[bf], sem_p
                ).wait()
                @pl.when(t >= NBUF)
                def _():
                    pltpu.make_async_copy(
                        res_v.at[bf], out_hbm.at[:, pl.ds(0, S_C)], sem_o
                    ).wait()

                def row_body(i, _):
                    @plsc.parallel_loop(0, HIDDEN, 16, unroll=2)
                    def _(o):
                        sl = pl.ds(pl.multiple_of(o, 16), 16)
                        p = pos_v[bf, i, sl]
                        for b in range(BATCH):
                            res_v[bf, b, i, sl] = (
                                rows_v[bf, b * S_C + i, sl] * SCALE + p
                            )

                    return 0

                lax.fori_loop(0, S_C, row_body, 0)

                pltpu.async_copy(
                    res_v.at[bf],
                    out_hbm.at[:, pl.ds(s_base + t * S_C, S_C)], sem_o,
                )

                @pl.when(t + NBUF < NCHUNK)
                def _():
                    issue_in(t + NBUF, bf)

            return 0

        lax.fori_loop(0, NCHUNK // NBUF, outer, 0)

        for bf in range(NBUF):
            pltpu.make_async_copy(
                res_v.at[bf], out_hbm.at[:, pl.ds(0, S_C)], sem_o
            ).wait()

    return k(ids, table, pos_emb)


def kernel(input_ids, table, pos_emb):
    ids = (
        input_ids.astype(jnp.int32)
        .reshape(BATCH, NW, NCHUNK, S_C)
        .transpose(1, 2, 0, 3)
        .reshape(-1)
    )
    return _sc_embed(ids, table, pos_emb)
